# Initial kernel scaffold; baseline (speedup 1.0000x reference)
import jax
import jax.numpy as jnp
from jax.experimental import pallas as pl


def kernel(x, weight, cluster_size, embed_avg):
    raise NotImplementedError("write your pallas kernel here")


# speedup vs baseline: 28.9583x; 28.9583x over previous
import jax
import jax.numpy as jnp
from jax.experimental import pallas as pl

B = 1024
D = 256
K = 1024
GAMMA = 0.99
EPS = 1e-05


def _tc_main(x_ref, w_ref, cs_ref, ea_ref, nw_ref, ncs_ref, nea_ref, am_ref):
    x = x_ref[...]
    w = w_ref[...]
    wsq = jnp.sum(w * w, axis=0, keepdims=True)
    xw = jax.lax.dot_general(
        x, w, (((1,), (0,)), ((), ())),
        preferred_element_type=jnp.float32,
        precision=jax.lax.Precision.HIGHEST,
    )
    scores = wsq - 2.0 * xw
    am = jnp.argmin(scores, axis=1).astype(jnp.int32)
    onehot = (jax.lax.broadcasted_iota(jnp.int32, (B, K), 1) == am[:, None]).astype(jnp.float32)
    counts = jnp.sum(onehot, axis=0)
    embed_sum = jax.lax.dot_general(
        x, onehot, (((0,), (0,)), ((), ())),
        preferred_element_type=jnp.float32,
        precision=jax.lax.Precision.HIGHEST,
    )
    n_idx = jnp.where(counts == 0.0, 1.0, counts)
    ncs = cs_ref[...] * GAMMA + (1.0 - GAMMA) * n_idx
    nea = ea_ref[...] * GAMMA + (1.0 - GAMMA) * embed_sum
    n = jnp.sum(ncs)
    cs_norm = (ncs + EPS) / (n + K * EPS) * n
    nw_ref[...] = nea / cs_norm[None, :]
    ncs_ref[...] = ncs
    nea_ref[...] = nea
    am_ref[...] = am


def kernel(x, weight, cluster_size, embed_avg):
    out_shapes = (
        jax.ShapeDtypeStruct((D, K), jnp.float32),
        jax.ShapeDtypeStruct((K,), jnp.float32),
        jax.ShapeDtypeStruct((D, K), jnp.float32),
        jax.ShapeDtypeStruct((B,), jnp.int32),
    )
    return pl.pallas_call(
        _tc_main,
        out_shape=out_shapes,
    )(x, weight, cluster_size, embed_avg)
